# Initial kernel scaffold; baseline (speedup 1.0000x reference)
import jax
import jax.numpy as jnp
from jax.experimental import pallas as pl


def kernel(x, Wg, W1, b1, W2, b2):
    raise NotImplementedError("write your pallas kernel here")


# speedup vs baseline: 1.9562x; 1.9562x over previous
import functools
import math

import jax
import jax.numpy as jnp
from jax import lax
from jax.experimental import pallas as pl
from jax.experimental.pallas import tpu as pltpu
from jax.experimental.pallas import tpu_sc as plsc

D_MODEL = 1024
D_HIDDEN = 4096
N_EXPERTS = 8
BT = 4096
CAP = 640
NUM_SLOTS = N_EXPERTS * CAP
XS_ROWS = NUM_SLOTS + 8
YE_ROWS = BT + 8

TOK_BLK = 1024
RANK_BLK = 512
HID_BLK = 512
NW = 32
TPW = BT // NW
CHUNK = 32


def _gate_body(x_ref, wg_ref, s_ref, e_ref):
    xb = x_ref[...]
    wg = wg_ref[...]
    lg = lax.dot_general(wg, xb, (((1,), (1,)), ((), ())),
                         preferred_element_type=jnp.float32)
    m = jnp.max(lg, axis=0, keepdims=True)
    p = jnp.exp(lg - m)
    s = jnp.sum(p, axis=0, keepdims=True)
    probs = p / s
    s_ref[...] = jnp.max(probs, axis=0, keepdims=True)
    e_ref[...] = jnp.argmax(probs, axis=0, keepdims=True).astype(jnp.int32)


def _gate(x_flat, wg):
    return pl.pallas_call(
        _gate_body,
        grid=(BT // TOK_BLK,),
        in_specs=[
            pl.BlockSpec((TOK_BLK, D_MODEL), lambda i: (i, 0)),
            pl.BlockSpec((N_EXPERTS, D_MODEL), lambda i: (0, 0)),
        ],
        out_specs=[
            pl.BlockSpec((1, TOK_BLK), lambda i: (0, i)),
            pl.BlockSpec((1, TOK_BLK), lambda i: (0, i)),
        ],
        out_shape=[
            jax.ShapeDtypeStruct((1, BT), jnp.float32),
            jax.ShapeDtypeStruct((1, BT), jnp.int32),
        ],
    )(x_flat, wg)


def _rank_body(s_row_ref, e_row_ref, s_col_ref, e_col_ref,
               ds_ref, dg_ref, yi_ref, lb_ref, ov_ref):
    i = pl.program_id(0)
    sc = s_col_ref[...]
    ec = e_col_ref[...]
    colidx = i * RANK_BLK + lax.broadcasted_iota(jnp.int32, (RANK_BLK, 1), 0)
    cnt = jnp.zeros((RANK_BLK, 1), jnp.float32)
    for c in range(BT // RANK_BLK):
        sr = s_row_ref[:, c * RANK_BLK:(c + 1) * RANK_BLK]
        er = e_row_ref[:, c * RANK_BLK:(c + 1) * RANK_BLK]
        rowidx = c * RANK_BLK + lax.broadcasted_iota(
            jnp.int32, (1, RANK_BLK), 1)
        same = er == ec
        beat = (sr > sc) | ((sr == sc) & (rowidx < colidx))
        cnt = cnt + jnp.sum(jnp.where(same & beat, 1.0, 0.0),
                            axis=1, keepdims=True)
    rank = cnt.astype(jnp.int32)
    kept = rank < CAP
    dest = jnp.where(kept, ec * CAP + rank, NUM_SLOTS)
    ds_ref[...] = dest
    dg_ref[...] = jnp.where(kept, dest, 0)
    yi_ref[...] = jnp.where(kept, colidx, BT)

    @pl.when(i == 0)
    def _():
        er_all = e_row_ref[...]
        eix = lax.broadcasted_iota(jnp.int32, (N_EXPERTS, 1), 0)
        counts = jnp.sum(jnp.where(er_all == eix, 1.0, 0.0),
                         axis=1, keepdims=True)
        expected = float(BT) / N_EXPERTS
        lb = (jnp.sum((counts - expected) ** 2, axis=0, keepdims=True)
              / N_EXPERTS) / (expected * expected)
        ov = jnp.sum(jnp.maximum(counts - float(CAP), 0.0),
                     axis=0, keepdims=True) / float(BT)
        lb_ref[...] = lb
        ov_ref[...] = ov


def _rank(s_row, e_row, s_col, e_col):
    nblk = BT // RANK_BLK
    return pl.pallas_call(
        _rank_body,
        grid=(nblk,),
        in_specs=[
            pl.BlockSpec((1, BT), lambda i: (0, 0)),
            pl.BlockSpec((1, BT), lambda i: (0, 0)),
            pl.BlockSpec((RANK_BLK, 1), lambda i: (i, 0)),
            pl.BlockSpec((RANK_BLK, 1), lambda i: (i, 0)),
        ],
        out_specs=[
            pl.BlockSpec((RANK_BLK, 1), lambda i: (i, 0)),
            pl.BlockSpec((RANK_BLK, 1), lambda i: (i, 0)),
            pl.BlockSpec((RANK_BLK, 1), lambda i: (i, 0)),
            pl.BlockSpec((1, 1), lambda i: (0, 0)),
            pl.BlockSpec((1, 1), lambda i: (0, 0)),
        ],
        out_shape=[
            jax.ShapeDtypeStruct((BT, 1), jnp.int32),
            jax.ShapeDtypeStruct((BT, 1), jnp.int32),
            jax.ShapeDtypeStruct((BT, 1), jnp.int32),
            jax.ShapeDtypeStruct((1, 1), jnp.float32),
            jax.ShapeDtypeStruct((1, 1), jnp.float32),
        ],
    )(s_row, e_row, s_col, e_col)


@functools.cache
def _sc_scatter_kernel():
    mesh = plsc.VectorSubcoreMesh(core_axis_name="c", subcore_axis_name="s")

    @functools.partial(
        pl.kernel,
        mesh=mesh,
        out_type=jax.ShapeDtypeStruct((XS_ROWS, D_MODEL), jnp.float32),
        scratch_types=[
            pltpu.VMEM((CHUNK,), jnp.int32),
            pltpu.VMEM((CHUNK, D_MODEL), jnp.float32),
            pltpu.SemaphoreType.DMA,
        ],
    )
    def _sc_scatter(x_hbm, ds_hbm, xs_hbm, idx_v, rows_v, sem):
        wid = lax.axis_index("s") * 2 + lax.axis_index("c")
        base = wid * TPW
        for cstart in range(0, TPW, CHUNK):
            sl = pl.ds(base + cstart, CHUNK)
            pltpu.sync_copy(ds_hbm.at[sl], idx_v)
            pltpu.sync_copy(x_hbm.at[sl], rows_v)
            pltpu.async_copy(rows_v, xs_hbm.at[idx_v], sem).wait()

    return _sc_scatter


def _ffn_body(xs_ref, w1_ref, b1_ref, w2_ref, b2_ref, ys_ref):
    hk = pl.program_id(1)
    xb = xs_ref[...]
    w1 = w1_ref[0]
    h = lax.dot_general(xb, w1, (((1,), (1,)), ((), ())),
                        preferred_element_type=jnp.float32)
    h = jax.nn.gelu(h + b1_ref[0])
    w2 = w2_ref[0]
    part = lax.dot_general(h, w2, (((1,), (1,)), ((), ())),
                           preferred_element_type=jnp.float32)

    @pl.when(hk == 0)
    def _():
        ys_ref[...] = part + b2_ref[0]

    @pl.when(hk != 0)
    def _():
        ys_ref[...] = ys_ref[...] + part


def _ffn(xs, w1, b1, w2, b2):
    nh = D_HIDDEN // HID_BLK
    return pl.pallas_call(
        _ffn_body,
        grid=(N_EXPERTS, nh),
        in_specs=[
            pl.BlockSpec((CAP, D_MODEL), lambda e, hk: (e, 0)),
            pl.BlockSpec((1, HID_BLK, D_MODEL), lambda e, hk: (e, hk, 0)),
            pl.BlockSpec((1, 1, HID_BLK), lambda e, hk: (e, 0, hk)),
            pl.BlockSpec((1, D_MODEL, HID_BLK), lambda e, hk: (e, 0, hk)),
            pl.BlockSpec((1, 1, D_MODEL), lambda e, hk: (e, 0, 0)),
        ],
        out_specs=pl.BlockSpec((CAP, D_MODEL), lambda e, hk: (e, 0)),
        out_shape=jax.ShapeDtypeStruct((NUM_SLOTS, D_MODEL), jnp.float32),
        compiler_params=pltpu.CompilerParams(
            dimension_semantics=("parallel", "arbitrary")),
    )(xs, w1, b1.reshape(N_EXPERTS, 1, D_HIDDEN),
      w2, b2.reshape(N_EXPERTS, 1, D_MODEL))


@functools.cache
def _sc_gather_kernel():
    mesh = plsc.VectorSubcoreMesh(core_axis_name="c", subcore_axis_name="s")

    @functools.partial(
        pl.kernel,
        mesh=mesh,
        out_type=jax.ShapeDtypeStruct((YE_ROWS, D_MODEL), jnp.float32),
        scratch_types=[
            pltpu.VMEM((CHUNK,), jnp.int32),
            pltpu.VMEM((CHUNK,), jnp.int32),
            pltpu.VMEM((CHUNK, D_MODEL), jnp.float32),
            pltpu.VMEM((CHUNK, D_MODEL), jnp.float32),
            pltpu.SemaphoreType.DMA,
            pltpu.SemaphoreType.DMA,
        ],
    )
    def _sc_gather(x_hbm, ys_hbm, dg_hbm, yi_hbm, y_hbm,
                   gi_v, si_v, xrows_v, yrows_v, sem1, sem2):
        wid = lax.axis_index("s") * 2 + lax.axis_index("c")
        base = wid * TPW
        for cstart in range(0, TPW, CHUNK):
            sl = pl.ds(base + cstart, CHUNK)
            pltpu.sync_copy(dg_hbm.at[sl], gi_v)
            pltpu.sync_copy(yi_hbm.at[sl], si_v)
            pltpu.sync_copy(x_hbm.at[sl], xrows_v)
            pltpu.sync_copy(xrows_v, y_hbm.at[sl])
            pltpu.async_copy(ys_hbm.at[gi_v], yrows_v, sem1).wait()
            pltpu.async_copy(yrows_v, y_hbm.at[si_v], sem2).wait()

    return _sc_gather


def kernel(x, Wg, W1, b1, W2, b2):
    Bx, Tx, D = x.shape
    x_flat = x.reshape(BT, D)
    scores, eids = _gate(x_flat, Wg)
    s_col = scores.reshape(BT, 1)
    e_col = eids.reshape(BT, 1)
    ds, dg, yi, lb, ov = _rank(scores, eids, s_col, e_col)
    xs = _sc_scatter_kernel()(x_flat, ds.reshape(BT))
    ys = _ffn(xs, W1, b1, W2, b2)
    y_ext = _sc_gather_kernel()(x_flat, ys, dg.reshape(BT), yi.reshape(BT))
    y = y_ext[:BT].reshape(Bx, Tx, D)
    return y, lb[0, 0], ov[0, 0]


# speedup vs baseline: 2.3507x; 1.2017x over previous
import functools
import math

import jax
import jax.numpy as jnp
from jax import lax
from jax.experimental import pallas as pl
from jax.experimental.pallas import tpu as pltpu
from jax.experimental.pallas import tpu_sc as plsc

D_MODEL = 1024
D_HIDDEN = 4096
N_EXPERTS = 8
BT = 4096
CAP = 640
NUM_SLOTS = N_EXPERTS * CAP
XS_ROWS = NUM_SLOTS + 8
YE_ROWS = BT + 8

TOK_BLK = 1024
RANK_BLK = 512
HID_BLK = 512
NW = 32
TPW = BT // NW
CHUNK = 32


def _gate_body(x_ref, wg_ref, s_ref, e_ref):
    xb = x_ref[...]
    wg = wg_ref[...]
    lg = lax.dot_general(wg, xb, (((1,), (1,)), ((), ())),
                         preferred_element_type=jnp.float32)
    m = jnp.max(lg, axis=0, keepdims=True)
    p = jnp.exp(lg - m)
    s = jnp.sum(p, axis=0, keepdims=True)
    probs = p / s
    s_ref[...] = jnp.max(probs, axis=0, keepdims=True)
    e_ref[...] = jnp.argmax(probs, axis=0, keepdims=True).astype(jnp.int32)


def _gate(x_flat, wg):
    return pl.pallas_call(
        _gate_body,
        grid=(BT // TOK_BLK,),
        in_specs=[
            pl.BlockSpec((TOK_BLK, D_MODEL), lambda i: (i, 0)),
            pl.BlockSpec((N_EXPERTS, D_MODEL), lambda i: (0, 0)),
        ],
        out_specs=[
            pl.BlockSpec((1, TOK_BLK), lambda i: (0, i)),
            pl.BlockSpec((1, TOK_BLK), lambda i: (0, i)),
        ],
        out_shape=[
            jax.ShapeDtypeStruct((1, BT), jnp.float32),
            jax.ShapeDtypeStruct((1, BT), jnp.int32),
        ],
    )(x_flat, wg)


def _rank_body(s_row_ref, e_row_ref, s_col_ref, e_col_ref,
               ds_ref, dg_ref, yi_ref, lb_ref, ov_ref):
    i = pl.program_id(0)
    sc = s_col_ref[...]
    ec = e_col_ref[...]
    colidx = i * RANK_BLK + lax.broadcasted_iota(jnp.int32, (RANK_BLK, 1), 0)
    cnt = jnp.zeros((RANK_BLK, 1), jnp.float32)
    for c in range(BT // RANK_BLK):
        sr = s_row_ref[:, c * RANK_BLK:(c + 1) * RANK_BLK]
        er = e_row_ref[:, c * RANK_BLK:(c + 1) * RANK_BLK]
        rowidx = c * RANK_BLK + lax.broadcasted_iota(
            jnp.int32, (1, RANK_BLK), 1)
        same = er == ec
        beat = (sr > sc) | ((sr == sc) & (rowidx < colidx))
        cnt = cnt + jnp.sum(jnp.where(same & beat, 1.0, 0.0),
                            axis=1, keepdims=True)
    rank = cnt.astype(jnp.int32)
    kept = rank < CAP
    dest = jnp.where(kept, ec * CAP + rank, NUM_SLOTS)
    ds_ref[...] = dest
    dg_ref[...] = jnp.where(kept, dest, 0)
    yi_ref[...] = jnp.where(kept, colidx, BT)

    @pl.when(i == 0)
    def _():
        er_all = e_row_ref[...]
        eix = lax.broadcasted_iota(jnp.int32, (N_EXPERTS, 1), 0)
        counts = jnp.sum(jnp.where(er_all == eix, 1.0, 0.0),
                         axis=1, keepdims=True)
        expected = float(BT) / N_EXPERTS
        lb = (jnp.sum((counts - expected) ** 2, axis=0, keepdims=True)
              / N_EXPERTS) / (expected * expected)
        ov = jnp.sum(jnp.maximum(counts - float(CAP), 0.0),
                     axis=0, keepdims=True) / float(BT)
        lb_ref[...] = lb
        ov_ref[...] = ov


def _rank(s_row, e_row, s_col, e_col):
    nblk = BT // RANK_BLK
    return pl.pallas_call(
        _rank_body,
        grid=(nblk,),
        in_specs=[
            pl.BlockSpec((1, BT), lambda i: (0, 0)),
            pl.BlockSpec((1, BT), lambda i: (0, 0)),
            pl.BlockSpec((RANK_BLK, 1), lambda i: (i, 0)),
            pl.BlockSpec((RANK_BLK, 1), lambda i: (i, 0)),
        ],
        out_specs=[
            pl.BlockSpec((RANK_BLK, 1), lambda i: (i, 0)),
            pl.BlockSpec((RANK_BLK, 1), lambda i: (i, 0)),
            pl.BlockSpec((RANK_BLK, 1), lambda i: (i, 0)),
            pl.BlockSpec((1, 1), lambda i: (0, 0)),
            pl.BlockSpec((1, 1), lambda i: (0, 0)),
        ],
        out_shape=[
            jax.ShapeDtypeStruct((BT, 1), jnp.int32),
            jax.ShapeDtypeStruct((BT, 1), jnp.int32),
            jax.ShapeDtypeStruct((BT, 1), jnp.int32),
            jax.ShapeDtypeStruct((1, 1), jnp.float32),
            jax.ShapeDtypeStruct((1, 1), jnp.float32),
        ],
    )(s_row, e_row, s_col, e_col)


@functools.cache
def _sc_scatter_kernel():
    mesh = plsc.VectorSubcoreMesh(core_axis_name="c", subcore_axis_name="s")

    @functools.partial(
        pl.kernel,
        mesh=mesh,
        out_type=jax.ShapeDtypeStruct((XS_ROWS, D_MODEL), jnp.float32),
        scratch_types=[
            pltpu.VMEM((CHUNK,), jnp.int32),
            pltpu.VMEM((CHUNK, D_MODEL), jnp.float32),
            pltpu.SemaphoreType.DMA,
        ],
    )
    def _sc_scatter(x_hbm, ds_hbm, xs_hbm, idx_v, rows_v, sem):
        wid = lax.axis_index("s") * 2 + lax.axis_index("c")
        base = wid * TPW
        for cstart in range(0, TPW, CHUNK):
            sl = pl.ds(base + cstart, CHUNK)
            pltpu.sync_copy(ds_hbm.at[sl], idx_v)
            pltpu.sync_copy(x_hbm.at[sl], rows_v)
            pltpu.async_copy(rows_v, xs_hbm.at[idx_v], sem).wait()

    return _sc_scatter


def _ffn_body(xs_ref, w1_ref, b1_ref, w2_ref, b2_ref, ys_ref):
    hk = pl.program_id(1)
    xb = xs_ref[...].astype(jnp.bfloat16)
    w1 = w1_ref[0].astype(jnp.bfloat16)
    h = lax.dot_general(xb, w1, (((1,), (1,)), ((), ())),
                        preferred_element_type=jnp.float32)
    h = jax.nn.gelu(h + b1_ref[0])
    w2 = w2_ref[0].astype(jnp.bfloat16)
    part = lax.dot_general(h.astype(jnp.bfloat16), w2,
                           (((1,), (1,)), ((), ())),
                           preferred_element_type=jnp.float32)

    @pl.when(hk == 0)
    def _():
        ys_ref[...] = part + b2_ref[0]

    @pl.when(hk != 0)
    def _():
        ys_ref[...] = ys_ref[...] + part


def _ffn(xs, w1, b1, w2, b2):
    nh = D_HIDDEN // HID_BLK
    return pl.pallas_call(
        _ffn_body,
        grid=(N_EXPERTS, nh),
        in_specs=[
            pl.BlockSpec((CAP, D_MODEL), lambda e, hk: (e, 0)),
            pl.BlockSpec((1, HID_BLK, D_MODEL), lambda e, hk: (e, hk, 0)),
            pl.BlockSpec((1, 1, HID_BLK), lambda e, hk: (e, 0, hk)),
            pl.BlockSpec((1, D_MODEL, HID_BLK), lambda e, hk: (e, 0, hk)),
            pl.BlockSpec((1, 1, D_MODEL), lambda e, hk: (e, 0, 0)),
        ],
        out_specs=pl.BlockSpec((CAP, D_MODEL), lambda e, hk: (e, 0)),
        out_shape=jax.ShapeDtypeStruct((NUM_SLOTS, D_MODEL), jnp.float32),
        compiler_params=pltpu.CompilerParams(
            dimension_semantics=("parallel", "arbitrary")),
    )(xs, w1, b1.reshape(N_EXPERTS, 1, D_HIDDEN),
      w2, b2.reshape(N_EXPERTS, 1, D_MODEL))


@functools.cache
def _sc_gather_kernel():
    mesh = plsc.VectorSubcoreMesh(core_axis_name="c", subcore_axis_name="s")

    @functools.partial(
        pl.kernel,
        mesh=mesh,
        out_type=jax.ShapeDtypeStruct((YE_ROWS, D_MODEL), jnp.float32),
        scratch_types=[
            pltpu.VMEM((CHUNK,), jnp.int32),
            pltpu.VMEM((CHUNK,), jnp.int32),
            pltpu.VMEM((CHUNK, D_MODEL), jnp.float32),
            pltpu.VMEM((CHUNK, D_MODEL), jnp.float32),
            pltpu.SemaphoreType.DMA,
            pltpu.SemaphoreType.DMA,
        ],
    )
    def _sc_gather(x_hbm, ys_hbm, dg_hbm, yi_hbm, y_hbm,
                   gi_v, si_v, xrows_v, yrows_v, sem1, sem2):
        wid = lax.axis_index("s") * 2 + lax.axis_index("c")
        base = wid * TPW
        for cstart in range(0, TPW, CHUNK):
            sl = pl.ds(base + cstart, CHUNK)
            pltpu.sync_copy(dg_hbm.at[sl], gi_v)
            pltpu.sync_copy(yi_hbm.at[sl], si_v)
            pltpu.sync_copy(x_hbm.at[sl], xrows_v)
            pltpu.sync_copy(xrows_v, y_hbm.at[sl])
            pltpu.async_copy(ys_hbm.at[gi_v], yrows_v, sem1).wait()
            pltpu.async_copy(yrows_v, y_hbm.at[si_v], sem2).wait()

    return _sc_gather


def kernel(x, Wg, W1, b1, W2, b2):
    Bx, Tx, D = x.shape
    x_flat = x.reshape(BT, D)
    scores, eids = _gate(x_flat, Wg)
    s_col = scores.reshape(BT, 1)
    e_col = eids.reshape(BT, 1)
    ds, dg, yi, lb, ov = _rank(scores, eids, s_col, e_col)
    xs = _sc_scatter_kernel()(x_flat, ds.reshape(BT))
    ys = _ffn(xs, W1, b1, W2, b2)
    y_ext = _sc_gather_kernel()(x_flat, ys, dg.reshape(BT), yi.reshape(BT))
    y = y_ext[:BT].reshape(Bx, Tx, D)
    return y, lb[0, 0], ov[0, 0]


# speedup vs baseline: 2.5399x; 1.0805x over previous
import functools
import math

import jax
import jax.numpy as jnp
from jax import lax
from jax.experimental import pallas as pl
from jax.experimental.pallas import tpu as pltpu
from jax.experimental.pallas import tpu_sc as plsc

D_MODEL = 1024
D_HIDDEN = 4096
N_EXPERTS = 8
BT = 4096
CAP = 640
NUM_SLOTS = N_EXPERTS * CAP
XS_ROWS = NUM_SLOTS + 8
YE_ROWS = BT + 8

TOK_BLK = 1024
RANK_BLK = 512
HID_BLK = 1024
NW = 32
TPW = BT // NW
CHUNK = 32


def _gate_body(x_ref, wg_ref, s_ref, e_ref):
    xb = x_ref[...]
    wg = wg_ref[...]
    lg = lax.dot_general(wg, xb, (((1,), (1,)), ((), ())),
                         preferred_element_type=jnp.float32)
    m = jnp.max(lg, axis=0, keepdims=True)
    p = jnp.exp(lg - m)
    s = jnp.sum(p, axis=0, keepdims=True)
    probs = p / s
    s_ref[...] = jnp.max(probs, axis=0, keepdims=True)
    e_ref[...] = jnp.argmax(probs, axis=0, keepdims=True).astype(jnp.int32)


def _gate(x_flat, wg):
    return pl.pallas_call(
        _gate_body,
        grid=(BT // TOK_BLK,),
        in_specs=[
            pl.BlockSpec((TOK_BLK, D_MODEL), lambda i: (i, 0)),
            pl.BlockSpec((N_EXPERTS, D_MODEL), lambda i: (0, 0)),
        ],
        out_specs=[
            pl.BlockSpec((1, TOK_BLK), lambda i: (0, i)),
            pl.BlockSpec((1, TOK_BLK), lambda i: (0, i)),
        ],
        out_shape=[
            jax.ShapeDtypeStruct((1, BT), jnp.float32),
            jax.ShapeDtypeStruct((1, BT), jnp.int32),
        ],
    )(x_flat, wg)


def _rank_body(s_row_ref, e_row_ref, s_col_ref, e_col_ref,
               ds_ref, dg_ref, yi_ref, lb_ref, ov_ref):
    i = pl.program_id(0)
    sc = s_col_ref[...]
    ec = e_col_ref[...]
    colidx = i * RANK_BLK + lax.broadcasted_iota(jnp.int32, (RANK_BLK, 1), 0)
    cnt = jnp.zeros((RANK_BLK, 1), jnp.float32)
    for c in range(BT // RANK_BLK):
        sr = s_row_ref[:, c * RANK_BLK:(c + 1) * RANK_BLK]
        er = e_row_ref[:, c * RANK_BLK:(c + 1) * RANK_BLK]
        rowidx = c * RANK_BLK + lax.broadcasted_iota(
            jnp.int32, (1, RANK_BLK), 1)
        same = er == ec
        beat = (sr > sc) | ((sr == sc) & (rowidx < colidx))
        cnt = cnt + jnp.sum(jnp.where(same & beat, 1.0, 0.0),
                            axis=1, keepdims=True)
    rank = cnt.astype(jnp.int32)
    kept = rank < CAP
    dest = jnp.where(kept, ec * CAP + rank, NUM_SLOTS)
    ds_ref[...] = dest
    dg_ref[...] = jnp.where(kept, dest, 0)
    yi_ref[...] = jnp.where(kept, colidx, BT)

    @pl.when(i == 0)
    def _():
        er_all = e_row_ref[...]
        eix = lax.broadcasted_iota(jnp.int32, (N_EXPERTS, 1), 0)
        counts = jnp.sum(jnp.where(er_all == eix, 1.0, 0.0),
                         axis=1, keepdims=True)
        expected = float(BT) / N_EXPERTS
        lb = (jnp.sum((counts - expected) ** 2, axis=0, keepdims=True)
              / N_EXPERTS) / (expected * expected)
        ov = jnp.sum(jnp.maximum(counts - float(CAP), 0.0),
                     axis=0, keepdims=True) / float(BT)
        lb_ref[...] = lb
        ov_ref[...] = ov


def _rank(s_row, e_row, s_col, e_col):
    nblk = BT // RANK_BLK
    return pl.pallas_call(
        _rank_body,
        grid=(nblk,),
        in_specs=[
            pl.BlockSpec((1, BT), lambda i: (0, 0)),
            pl.BlockSpec((1, BT), lambda i: (0, 0)),
            pl.BlockSpec((RANK_BLK, 1), lambda i: (i, 0)),
            pl.BlockSpec((RANK_BLK, 1), lambda i: (i, 0)),
        ],
        out_specs=[
            pl.BlockSpec((RANK_BLK, 1), lambda i: (i, 0)),
            pl.BlockSpec((RANK_BLK, 1), lambda i: (i, 0)),
            pl.BlockSpec((RANK_BLK, 1), lambda i: (i, 0)),
            pl.BlockSpec((1, 1), lambda i: (0, 0)),
            pl.BlockSpec((1, 1), lambda i: (0, 0)),
        ],
        out_shape=[
            jax.ShapeDtypeStruct((BT, 1), jnp.int32),
            jax.ShapeDtypeStruct((BT, 1), jnp.int32),
            jax.ShapeDtypeStruct((BT, 1), jnp.int32),
            jax.ShapeDtypeStruct((1, 1), jnp.float32),
            jax.ShapeDtypeStruct((1, 1), jnp.float32),
        ],
    )(s_row, e_row, s_col, e_col)


@functools.cache
def _sc_scatter_kernel():
    mesh = plsc.VectorSubcoreMesh(core_axis_name="c", subcore_axis_name="s")

    @functools.partial(
        pl.kernel,
        mesh=mesh,
        out_type=jax.ShapeDtypeStruct((XS_ROWS, D_MODEL), jnp.float32),
        scratch_types=[
            pltpu.VMEM((CHUNK,), jnp.int32),
            pltpu.VMEM((CHUNK, D_MODEL), jnp.float32),
            pltpu.SemaphoreType.DMA,
        ],
    )
    def _sc_scatter(x_hbm, ds_hbm, xs_hbm, idx_v, rows_v, sem):
        wid = lax.axis_index("s") * 2 + lax.axis_index("c")
        base = wid * TPW
        for cstart in range(0, TPW, CHUNK):
            sl = pl.ds(base + cstart, CHUNK)
            pltpu.sync_copy(ds_hbm.at[sl], idx_v)
            pltpu.sync_copy(x_hbm.at[sl], rows_v)
            pltpu.async_copy(rows_v, xs_hbm.at[idx_v], sem).wait()

    return _sc_scatter


def _ffn_body(xs_ref, w1_ref, b1_ref, w2_ref, b2_ref, ys_ref, x16_ref):
    hk = pl.program_id(1)

    @pl.when(hk == 0)
    def _():
        x16_ref[...] = xs_ref[...].astype(jnp.bfloat16)

    xb = x16_ref[...]
    w1 = w1_ref[0].astype(jnp.bfloat16)
    h = lax.dot_general(xb, w1, (((1,), (1,)), ((), ())),
                        preferred_element_type=jnp.float32)
    h = jax.nn.gelu(h + b1_ref[0])
    w2 = w2_ref[0].astype(jnp.bfloat16)
    part = lax.dot_general(h.astype(jnp.bfloat16), w2,
                           (((1,), (1,)), ((), ())),
                           preferred_element_type=jnp.float32)

    @pl.when(hk == 0)
    def _():
        ys_ref[...] = part + b2_ref[0]

    @pl.when(hk != 0)
    def _():
        ys_ref[...] = ys_ref[...] + part


def _ffn(xs, w1, b1, w2, b2):
    nh = D_HIDDEN // HID_BLK
    return pl.pallas_call(
        _ffn_body,
        grid=(N_EXPERTS, nh),
        in_specs=[
            pl.BlockSpec((CAP, D_MODEL), lambda e, hk: (e, 0)),
            pl.BlockSpec((1, HID_BLK, D_MODEL), lambda e, hk: (e, hk, 0)),
            pl.BlockSpec((1, 1, HID_BLK), lambda e, hk: (e, 0, hk)),
            pl.BlockSpec((1, D_MODEL, HID_BLK), lambda e, hk: (e, 0, hk)),
            pl.BlockSpec((1, 1, D_MODEL), lambda e, hk: (e, 0, 0)),
        ],
        out_specs=pl.BlockSpec((CAP, D_MODEL), lambda e, hk: (e, 0)),
        out_shape=jax.ShapeDtypeStruct((NUM_SLOTS, D_MODEL), jnp.float32),
        scratch_shapes=[pltpu.VMEM((CAP, D_MODEL), jnp.bfloat16)],
        compiler_params=pltpu.CompilerParams(
            dimension_semantics=("arbitrary", "arbitrary")),
    )(xs, w1, b1.reshape(N_EXPERTS, 1, D_HIDDEN),
      w2, b2.reshape(N_EXPERTS, 1, D_MODEL))


@functools.cache
def _sc_gather_kernel():
    mesh = plsc.VectorSubcoreMesh(core_axis_name="c", subcore_axis_name="s")

    @functools.partial(
        pl.kernel,
        mesh=mesh,
        out_type=jax.ShapeDtypeStruct((YE_ROWS, D_MODEL), jnp.float32),
        scratch_types=[
            pltpu.VMEM((CHUNK,), jnp.int32),
            pltpu.VMEM((CHUNK,), jnp.int32),
            pltpu.VMEM((CHUNK, D_MODEL), jnp.float32),
            pltpu.VMEM((CHUNK, D_MODEL), jnp.float32),
            pltpu.SemaphoreType.DMA,
            pltpu.SemaphoreType.DMA,
        ],
    )
    def _sc_gather(x_hbm, ys_hbm, dg_hbm, yi_hbm, y_hbm,
                   gi_v, si_v, xrows_v, yrows_v, sem1, sem2):
        wid = lax.axis_index("s") * 2 + lax.axis_index("c")
        base = wid * TPW
        for cstart in range(0, TPW, CHUNK):
            sl = pl.ds(base + cstart, CHUNK)
            pltpu.sync_copy(dg_hbm.at[sl], gi_v)
            pltpu.sync_copy(yi_hbm.at[sl], si_v)
            pltpu.sync_copy(x_hbm.at[sl], xrows_v)
            pltpu.sync_copy(xrows_v, y_hbm.at[sl])
            pltpu.async_copy(ys_hbm.at[gi_v], yrows_v, sem1).wait()
            pltpu.async_copy(yrows_v, y_hbm.at[si_v], sem2).wait()

    return _sc_gather


def kernel(x, Wg, W1, b1, W2, b2):
    Bx, Tx, D = x.shape
    x_flat = x.reshape(BT, D)
    scores, eids = _gate(x_flat, Wg)
    s_col = scores.reshape(BT, 1)
    e_col = eids.reshape(BT, 1)
    ds, dg, yi, lb, ov = _rank(scores, eids, s_col, e_col)
    xs = _sc_scatter_kernel()(x_flat, ds.reshape(BT))
    ys = _ffn(xs, W1, b1, W2, b2)
    y_ext = _sc_gather_kernel()(x_flat, ys, dg.reshape(BT), yi.reshape(BT))
    y = y_ext[:BT].reshape(Bx, Tx, D)
    return y, lb[0, 0], ov[0, 0]


# speedup vs baseline: 2.8852x; 1.1360x over previous
import functools
import math

import jax
import jax.numpy as jnp
from jax import lax
from jax.experimental import pallas as pl
from jax.experimental.pallas import tpu as pltpu
from jax.experimental.pallas import tpu_sc as plsc

D_MODEL = 1024
D_HIDDEN = 4096
N_EXPERTS = 8
BT = 4096
CAP = 640
NUM_SLOTS = N_EXPERTS * CAP
DUMP_ROW = NUM_SLOTS
PASS_BASE = NUM_SLOTS + 8
BIG_ROWS = PASS_BASE + BT

TOK_BLK = 1024
RANK_BLK = 512
HID_BLK = 1024
NW = 32
TPW = BT // NW
CHUNK = 32


def _gate_body(x_ref, wg_ref, s_ref, e_ref):
    xb = x_ref[...]
    wg = wg_ref[...]
    lg = lax.dot_general(wg, xb, (((1,), (1,)), ((), ())),
                         preferred_element_type=jnp.float32)
    m = jnp.max(lg, axis=0, keepdims=True)
    p = jnp.exp(lg - m)
    s = jnp.sum(p, axis=0, keepdims=True)
    probs = p / s
    s_ref[...] = jnp.max(probs, axis=0, keepdims=True)
    e_ref[...] = jnp.argmax(probs, axis=0, keepdims=True).astype(jnp.int32)


def _gate(x_flat, wg):
    return pl.pallas_call(
        _gate_body,
        grid=(BT // TOK_BLK,),
        in_specs=[
            pl.BlockSpec((TOK_BLK, D_MODEL), lambda i: (i, 0)),
            pl.BlockSpec((N_EXPERTS, D_MODEL), lambda i: (0, 0)),
        ],
        out_specs=[
            pl.BlockSpec((1, TOK_BLK), lambda i: (0, i)),
            pl.BlockSpec((1, TOK_BLK), lambda i: (0, i)),
        ],
        out_shape=[
            jax.ShapeDtypeStruct((1, BT), jnp.float32),
            jax.ShapeDtypeStruct((1, BT), jnp.int32),
        ],
    )(x_flat, wg)


def _rank_body(s_row_ref, e_row_ref, s_col_ref, e_col_ref,
               ds_ref, dg_ref, lb_ref, ov_ref):
    i = pl.program_id(0)
    sc = s_col_ref[...]
    ec = e_col_ref[...]
    colidx = i * RANK_BLK + lax.broadcasted_iota(jnp.int32, (RANK_BLK, 1), 0)
    cnt = jnp.zeros((RANK_BLK, 1), jnp.float32)
    for c in range(BT // RANK_BLK):
        sr = s_row_ref[:, c * RANK_BLK:(c + 1) * RANK_BLK]
        er = e_row_ref[:, c * RANK_BLK:(c + 1) * RANK_BLK]
        rowidx = c * RANK_BLK + lax.broadcasted_iota(
            jnp.int32, (1, RANK_BLK), 1)
        same = er == ec
        beat = (sr > sc) | ((sr == sc) & (rowidx < colidx))
        cnt = cnt + jnp.sum(jnp.where(same & beat, 1.0, 0.0),
                            axis=1, keepdims=True)
    rank = cnt.astype(jnp.int32)
    kept = rank < CAP
    slot = ec * CAP + rank
    ds_ref[...] = jnp.where(kept, slot, DUMP_ROW)
    dg_ref[...] = jnp.where(kept, slot, PASS_BASE + colidx)

    @pl.when(i == 0)
    def _():
        er_all = e_row_ref[...]
        eix = lax.broadcasted_iota(jnp.int32, (N_EXPERTS, 1), 0)
        counts = jnp.sum(jnp.where(er_all == eix, 1.0, 0.0),
                         axis=1, keepdims=True)
        expected = float(BT) / N_EXPERTS
        lb = (jnp.sum((counts - expected) ** 2, axis=0, keepdims=True)
              / N_EXPERTS) / (expected * expected)
        ov = jnp.sum(jnp.maximum(counts - float(CAP), 0.0),
                     axis=0, keepdims=True) / float(BT)
        lb_ref[...] = lb
        ov_ref[...] = ov


def _rank(s_row, e_row, s_col, e_col):
    nblk = BT // RANK_BLK
    return pl.pallas_call(
        _rank_body,
        grid=(nblk,),
        in_specs=[
            pl.BlockSpec((1, BT), lambda i: (0, 0)),
            pl.BlockSpec((1, BT), lambda i: (0, 0)),
            pl.BlockSpec((RANK_BLK, 1), lambda i: (i, 0)),
            pl.BlockSpec((RANK_BLK, 1), lambda i: (i, 0)),
        ],
        out_specs=[
            pl.BlockSpec((RANK_BLK, 1), lambda i: (i, 0)),
            pl.BlockSpec((RANK_BLK, 1), lambda i: (i, 0)),
            pl.BlockSpec((1, 1), lambda i: (0, 0)),
            pl.BlockSpec((1, 1), lambda i: (0, 0)),
        ],
        out_shape=[
            jax.ShapeDtypeStruct((BT, 1), jnp.int32),
            jax.ShapeDtypeStruct((BT, 1), jnp.int32),
            jax.ShapeDtypeStruct((1, 1), jnp.float32),
            jax.ShapeDtypeStruct((1, 1), jnp.float32),
        ],
    )(s_row, e_row, s_col, e_col)


@functools.cache
def _sc_scatter_kernel():
    mesh = plsc.VectorSubcoreMesh(core_axis_name="c", subcore_axis_name="s")

    @functools.partial(
        pl.kernel,
        mesh=mesh,
        out_type=jax.ShapeDtypeStruct((BIG_ROWS, D_MODEL), jnp.float32),
        scratch_types=[
            pltpu.VMEM((CHUNK,), jnp.int32),
            pltpu.VMEM((CHUNK, D_MODEL), jnp.float32),
            pltpu.SemaphoreType.DMA,
            pltpu.SemaphoreType.DMA,
        ],
    )
    def _sc_scatter(x_hbm, ds_hbm, big_hbm, idx_v, rows_v, sem, sem2):
        wid = lax.axis_index("s") * 2 + lax.axis_index("c")
        base = wid * TPW
        for cstart in range(0, TPW, CHUNK):
            sl = pl.ds(base + cstart, CHUNK)
            pltpu.sync_copy(ds_hbm.at[sl], idx_v)
            pltpu.sync_copy(x_hbm.at[sl], rows_v)
            ps = pl.ds(PASS_BASE + base + cstart, CHUNK)
            cp = pltpu.async_copy(rows_v, big_hbm.at[ps], sem2)
            pltpu.async_copy(rows_v, big_hbm.at[idx_v], sem).wait()
            cp.wait()

    return _sc_scatter


def _ffn_body(xs_ref, w1_ref, b1_ref, w2_ref, b2_ref, ys_ref, x16_ref):
    hk = pl.program_id(1)

    @pl.when(hk == 0)
    def _():
        x16_ref[...] = xs_ref[...].astype(jnp.bfloat16)

    xb = x16_ref[...]
    w1 = w1_ref[0].astype(jnp.bfloat16)
    h = lax.dot_general(xb, w1, (((1,), (1,)), ((), ())),
                        preferred_element_type=jnp.float32)
    h = jax.nn.gelu(h + b1_ref[0])
    w2 = w2_ref[0].astype(jnp.bfloat16)
    part = lax.dot_general(h.astype(jnp.bfloat16), w2,
                           (((1,), (1,)), ((), ())),
                           preferred_element_type=jnp.float32)

    @pl.when(hk == 0)
    def _():
        ys_ref[...] = part + b2_ref[0]

    @pl.when(hk != 0)
    def _():
        ys_ref[...] = ys_ref[...] + part


def _ffn(xs, w1, b1, w2, b2):
    nh = D_HIDDEN // HID_BLK
    return pl.pallas_call(
        _ffn_body,
        grid=(N_EXPERTS, nh),
        in_specs=[
            pl.BlockSpec((CAP, D_MODEL), lambda e, hk: (e, 0)),
            pl.BlockSpec((1, HID_BLK, D_MODEL), lambda e, hk: (e, hk, 0)),
            pl.BlockSpec((1, 1, HID_BLK), lambda e, hk: (e, 0, hk)),
            pl.BlockSpec((1, D_MODEL, HID_BLK), lambda e, hk: (e, 0, hk)),
            pl.BlockSpec((1, 1, D_MODEL), lambda e, hk: (e, 0, 0)),
        ],
        out_specs=pl.BlockSpec((CAP, D_MODEL), lambda e, hk: (e, 0)),
        out_shape=jax.ShapeDtypeStruct((BIG_ROWS, D_MODEL), jnp.float32),
        scratch_shapes=[pltpu.VMEM((CAP, D_MODEL), jnp.bfloat16)],
        input_output_aliases={0: 0},
        compiler_params=pltpu.CompilerParams(
            dimension_semantics=("arbitrary", "arbitrary")),
    )(xs, w1, b1.reshape(N_EXPERTS, 1, D_HIDDEN),
      w2, b2.reshape(N_EXPERTS, 1, D_MODEL))


@functools.cache
def _sc_gather_kernel():
    mesh = plsc.VectorSubcoreMesh(core_axis_name="c", subcore_axis_name="s")

    @functools.partial(
        pl.kernel,
        mesh=mesh,
        out_type=jax.ShapeDtypeStruct((BT, D_MODEL), jnp.float32),
        scratch_types=[
            pltpu.VMEM((CHUNK,), jnp.int32),
            pltpu.VMEM((CHUNK, D_MODEL), jnp.float32),
            pltpu.SemaphoreType.DMA,
        ],
    )
    def _sc_gather(big_hbm, dg_hbm, y_hbm, gi_v, yrows_v, sem1):
        wid = lax.axis_index("s") * 2 + lax.axis_index("c")
        base = wid * TPW
        for cstart in range(0, TPW, CHUNK):
            sl = pl.ds(base + cstart, CHUNK)
            pltpu.sync_copy(dg_hbm.at[sl], gi_v)
            pltpu.async_copy(big_hbm.at[gi_v], yrows_v, sem1).wait()
            pltpu.sync_copy(yrows_v, y_hbm.at[sl])

    return _sc_gather


def kernel(x, Wg, W1, b1, W2, b2):
    Bx, Tx, D = x.shape
    x_flat = x.reshape(BT, D)
    scores, eids = _gate(x_flat, Wg)
    s_col = scores.reshape(BT, 1)
    e_col = eids.reshape(BT, 1)
    ds, dg, lb, ov = _rank(scores, eids, s_col, e_col)
    big = _sc_scatter_kernel()(x_flat, ds.reshape(BT))
    big = _ffn(big, W1, b1, W2, b2)
    y = _sc_gather_kernel()(big, dg.reshape(BT))
    return y.reshape(Bx, Tx, D), lb[0, 0], ov[0, 0]
